# baseline (device time: 15370 ns/iter reference)
import jax
import jax.numpy as jnp
from jax import lax
from jax.experimental import pallas as pl
from jax.experimental.pallas import tpu as pltpu

N_DEV = 4
B, SQ, HQ, DH = 2, 256, 4, 64
SKV = 1024 // N_DEV
D_MODEL = 512
QD = HQ * DH
BLK = 64
H2 = SQ // 2
PACKH = H2 + HQ


def _body(x_ref, wq_ref, k_ref, v_ref, wo_ref, out_ref,
          pack, recv1, pack2, recv2, ctx_ref,
          s1send, s1recv, s2send, s2recv):
    my = lax.axis_index("i")
    partner1 = jnp.bitwise_xor(my, 1)
    partner2 = (N_DEV - 1) - my
    ph1_partner = (partner1, partner2)
    ph2_partner = (partner2, partner1)

    barrier = pltpu.get_barrier_semaphore()
    for peer in (partner1, partner2):
        pl.semaphore_signal(barrier, inc=1, device_id=(peer,),
                            device_id_type=pl.DeviceIdType.MESH)

    wq_bf = wq_ref[...].astype(jnp.bfloat16)

    row_blk = lax.broadcasted_iota(jnp.int32, (SQ, SKV), 0) // BLK
    col_blk = lax.broadcasted_iota(jnp.int32, (SQ, SKV), 1) // BLK
    jblk = col_blk + my * (SKV // BLK)
    mask = ((row_blk == jblk) | (jblk == 0)
            | (lax.rem(row_blk + jblk, 3) == 0))

    ones_row = jnp.ones((1, SKV), jnp.bfloat16)
    ph1 = {}
    for b in range(B):
        kb = k_ref[b].astype(jnp.bfloat16)
        vb = v_ref[b].astype(jnp.bfloat16)
        for qh in range(2):
            r0 = b * SQ + qh * H2
            qhalf = jnp.dot(x_ref[r0:r0 + H2, :].astype(jnp.bfloat16),
                            wq_bf, preferred_element_type=jnp.float32)
            qhalf = (qhalf * 0.125).astype(jnp.bfloat16)
            mh = mask[qh * H2:(qh + 1) * H2, :]
            for h in range(HQ):
                qbh = qhalf[:, h * DH:(h + 1) * DH]
                s = lax.dot_general(
                    qbh, kb[:, h * DH:(h + 1) * DH],
                    (((1,), (1,)), ((), ())),
                    preferred_element_type=jnp.float32)
                p = jnp.where(mh, jnp.exp(s), 0.0).astype(jnp.bfloat16)
                pv = jnp.dot(p, vb[:, h * DH:(h + 1) * DH],
                             preferred_element_type=jnp.float32)
                l_row = lax.dot_general(
                    ones_row, p, (((1,), (1,)), ((), ())),
                    preferred_element_type=jnp.float32)
                pack[b, qh, 0:H2, h * DH:(h + 1) * DH] = (
                    pv.astype(jnp.bfloat16))
                pack[b, qh, H2 + h:H2 + h + 1, 0:H2] = (
                    l_row.astype(jnp.bfloat16))
            if b == 0 and qh == 0:
                pl.semaphore_wait(barrier, 2)
            rdma = pltpu.make_async_remote_copy(
                src_ref=pack.at[b, qh], dst_ref=recv1.at[b, qh],
                send_sem=s1send.at[b, qh], recv_sem=s1recv.at[b, qh],
                device_id=(ph1_partner[b],),
                device_id_type=pl.DeviceIdType.MESH)
            rdma.start()
            ph1[(b, qh)] = rdma

    eye = jnp.where(
        lax.broadcasted_iota(jnp.int32, (H2, H2), 0)
        == lax.broadcasted_iota(jnp.int32, (H2, H2), 1),
        1.0, 0.0).astype(jnp.bfloat16)

    ph2 = {}
    tots = {}
    for b in range(B):
        for qh in range(2):
            ph1[(b, qh)].wait_recv()
            tot = pack[b, qh] + recv1[b, qh]
            pack2[b, qh] = tot
            rdma = pltpu.make_async_remote_copy(
                src_ref=pack2.at[b, qh], dst_ref=recv2.at[b, qh],
                send_sem=s2send.at[b, qh], recv_sem=s2recv.at[b, qh],
                device_id=(ph2_partner[b],),
                device_id_type=pl.DeviceIdType.MESH)
            rdma.start()
            ph2[(b, qh)] = rdma
            tots[(b, qh)] = tot

    wo_bf = wo_ref[...].astype(jnp.bfloat16)
    for b in range(B):
        for qh in range(2):
            ph2[(b, qh)].wait_recv()
            tot = tots[(b, qh)] + recv2[b, qh]
            l_rows = tot[H2:PACKH, 0:H2]
            l_cols = lax.dot_general(eye, l_rows,
                                     (((1,), (1,)), ((), ())),
                                     preferred_element_type=jnp.float32)
            rcp = (1.0 / l_cols).astype(jnp.bfloat16)
            r0 = b * SQ + qh * H2
            for h in range(HQ):
                blk = tot[0:H2, h * DH:(h + 1) * DH]
                ctx_ref[r0:r0 + H2, h * DH:(h + 1) * DH] = (
                    blk * rcp[:, h:h + 1])
        out_ref[b * SQ:(b + 1) * SQ, :] = jnp.dot(
            ctx_ref[b * SQ:(b + 1) * SQ, :], wo_bf,
            preferred_element_type=jnp.float32).astype(jnp.bfloat16)

    for rdma in list(ph1.values()) + list(ph2.values()):
        rdma.wait_send()


def kernel(x, Wq, K_ext, V_ext, Wo):
    x2 = x.reshape(B * SQ, D_MODEL)
    k2 = K_ext.reshape(B, SKV, HQ * DH)
    v2 = V_ext.reshape(B, SKV, HQ * DH)

    out = pl.pallas_call(
        _body,
        out_shape=jax.ShapeDtypeStruct((B * SQ, D_MODEL), jnp.bfloat16),
        in_specs=[pl.BlockSpec(memory_space=pltpu.VMEM)] * 5,
        out_specs=pl.BlockSpec(memory_space=pltpu.VMEM),
        scratch_shapes=[
            pltpu.VMEM((B, 2, PACKH, QD), jnp.bfloat16),
            pltpu.VMEM((B, 2, PACKH, QD), jnp.bfloat16),
            pltpu.VMEM((B, 2, PACKH, QD), jnp.bfloat16),
            pltpu.VMEM((B, 2, PACKH, QD), jnp.bfloat16),
            pltpu.VMEM((B * SQ, QD), jnp.bfloat16),
            pltpu.SemaphoreType.DMA((B, 2)),
            pltpu.SemaphoreType.DMA((B, 2)),
            pltpu.SemaphoreType.DMA((B, 2)),
            pltpu.SemaphoreType.DMA((B, 2)),
        ],
        compiler_params=pltpu.CompilerParams(collective_id=0),
    )(x2, Wq, k2, v2, Wo)
    return out.reshape(B, SQ, D_MODEL)


# device time: 15267 ns/iter; 1.0067x vs baseline; 1.0067x over previous
import jax
import jax.numpy as jnp
from jax import lax
from jax.experimental import pallas as pl
from jax.experimental.pallas import tpu as pltpu

N_DEV = 4
B, SQ, HQ, DH = 2, 256, 4, 64
SKV = 1024 // N_DEV
D_MODEL = 512
QD = HQ * DH
BLK = 64
PACK = SQ + HQ


def _body(x_ref, wq_ref, k_ref, v_ref, wo_ref, out_ref,
          pack, recv1, pack2, recv2, ctx_ref,
          s1send, s1recv, s2send, s2recv):
    my = lax.axis_index("i")
    partner1 = jnp.bitwise_xor(my, 1)
    partner2 = (N_DEV - 1) - my
    ph1_partner = (partner1, partner2)
    ph2_partner = (partner2, partner1)

    barrier = pltpu.get_barrier_semaphore()
    for peer in (partner1, partner2):
        pl.semaphore_signal(barrier, inc=1, device_id=(peer,),
                            device_id_type=pl.DeviceIdType.MESH)

    wq_bf = wq_ref[...].astype(jnp.bfloat16)

    row_blk = lax.broadcasted_iota(jnp.int32, (SQ, SKV), 0) // BLK
    col_blk = lax.broadcasted_iota(jnp.int32, (SQ, SKV), 1) // BLK
    jblk = col_blk + my * (SKV // BLK)
    mask = ((row_blk == jblk) | (jblk == 0)
            | (lax.rem(row_blk + jblk, 3) == 0))

    ones_row = jnp.ones((1, SKV), jnp.bfloat16)
    ph1 = []
    for b in range(B):
        kb = k_ref[b].astype(jnp.bfloat16)
        vb = v_ref[b].astype(jnp.bfloat16)
        qb = jnp.dot(x_ref[b * SQ:(b + 1) * SQ, :].astype(jnp.bfloat16),
                     wq_bf, preferred_element_type=jnp.float32)
        qb = (qb * 0.125).astype(jnp.bfloat16)
        for h in range(HQ):
            qbh = qb[:, h * DH:(h + 1) * DH]
            s = lax.dot_general(
                qbh, kb[:, h * DH:(h + 1) * DH],
                (((1,), (1,)), ((), ())),
                preferred_element_type=jnp.float32)
            p = jnp.where(mask, jnp.exp(s), 0.0).astype(jnp.bfloat16)
            pv = jnp.dot(p, vb[:, h * DH:(h + 1) * DH],
                         preferred_element_type=jnp.float32)
            l_row = lax.dot_general(
                ones_row, p, (((1,), (1,)), ((), ())),
                preferred_element_type=jnp.float32)
            pack[b, 0:SQ, h * DH:(h + 1) * DH] = pv.astype(jnp.bfloat16)
            pack[b, SQ + h:SQ + h + 1, :] = l_row.astype(jnp.bfloat16)
        if b == 0:
            pl.semaphore_wait(barrier, 2)
        rdma = pltpu.make_async_remote_copy(
            src_ref=pack.at[b], dst_ref=recv1.at[b],
            send_sem=s1send.at[b], recv_sem=s1recv.at[b],
            device_id=(ph1_partner[b],), device_id_type=pl.DeviceIdType.MESH)
        rdma.start()
        ph1.append(rdma)

    eye = jnp.where(
        lax.broadcasted_iota(jnp.int32, (SQ, SQ), 0)
        == lax.broadcasted_iota(jnp.int32, (SQ, SQ), 1),
        1.0, 0.0).astype(jnp.bfloat16)

    ph2 = []
    tots = []
    for b in range(B):
        ph1[b].wait_recv()
        tot = pack[b] + recv1[b]
        pack2[b] = tot
        rdma = pltpu.make_async_remote_copy(
            src_ref=pack2.at[b], dst_ref=recv2.at[b],
            send_sem=s2send.at[b], recv_sem=s2recv.at[b],
            device_id=(ph2_partner[b],), device_id_type=pl.DeviceIdType.MESH)
        rdma.start()
        ph2.append(rdma)
        tots.append(tot)

    wo_bf = wo_ref[...].astype(jnp.bfloat16)
    for b in range(B):
        ph2[b].wait_recv()
        tot = tots[b] + recv2[b]
        l_rows = tot[SQ:PACK, :]
        l_cols = lax.dot_general(eye, l_rows, (((1,), (1,)), ((), ())),
                                 preferred_element_type=jnp.float32)
        rcp = (1.0 / l_cols).astype(jnp.bfloat16)
        for h in range(HQ):
            blk = tot[0:SQ, h * DH:(h + 1) * DH]
            ctx_ref[b * SQ:(b + 1) * SQ, h * DH:(h + 1) * DH] = (
                blk * rcp[:, h:h + 1])
        out_ref[b * SQ:(b + 1) * SQ, :] = jnp.dot(
            ctx_ref[b * SQ:(b + 1) * SQ, :], wo_bf,
            preferred_element_type=jnp.float32).astype(jnp.bfloat16)

    for rdma in ph1 + ph2:
        rdma.wait_send()


def kernel(x, Wq, K_ext, V_ext, Wo):
    x2 = x.reshape(B * SQ, D_MODEL)
    k2 = K_ext.reshape(B, SKV, HQ * DH)
    v2 = V_ext.reshape(B, SKV, HQ * DH)

    out = pl.pallas_call(
        _body,
        out_shape=jax.ShapeDtypeStruct((B * SQ, D_MODEL), jnp.bfloat16),
        in_specs=[pl.BlockSpec(memory_space=pltpu.VMEM)] * 5,
        out_specs=pl.BlockSpec(memory_space=pltpu.VMEM),
        scratch_shapes=[
            pltpu.VMEM((B, PACK, QD), jnp.bfloat16),
            pltpu.VMEM((B, PACK, QD), jnp.bfloat16),
            pltpu.VMEM((B, PACK, QD), jnp.bfloat16),
            pltpu.VMEM((B, PACK, QD), jnp.bfloat16),
            pltpu.VMEM((B * SQ, QD), jnp.bfloat16),
            pltpu.SemaphoreType.DMA((B,)),
            pltpu.SemaphoreType.DMA((B,)),
            pltpu.SemaphoreType.DMA((B,)),
            pltpu.SemaphoreType.DMA((B,)),
        ],
        compiler_params=pltpu.CompilerParams(collective_id=0),
    )(x2, Wq, k2, v2, Wo)
    return out.reshape(B, SQ, D_MODEL)
